# baseline (device time: 89645 ns/iter reference)
import jax
import jax.numpy as jnp
from jax import lax
from jax.experimental import pallas as pl
from jax.experimental.pallas import tpu as pltpu

N_DEV = 4
N_HOP = N_DEV - 1


def kernel(ids, E):
    T = ids.shape[0]
    V_per, D = E.shape
    H = T // 2
    R = H // N_DEV

    my = lax.axis_index("i")
    loc = ids - my * V_per
    mask = (loc >= 0) & (loc < V_per)
    safe = jnp.where(mask, loc, 0).astype(jnp.int32)
    mask_i = mask.astype(jnp.int32)
    maskf = mask.astype(jnp.bfloat16)[:, None]

    def body(safe_ref, mask_i_ref, maskf_ref, e_ref, out_ref, gbuf, red_ref,
             rs_buf, gsem, rs_send, rs_recv, ag_send, ag_recv):
        my_pos = lax.axis_index("i")
        left = lax.rem(my_pos + N_DEV - 1, N_DEV)
        right = lax.rem(my_pos + 1, N_DEV)
        peer = (right, left)

        def cidx(off):
            return lax.rem(my_pos + (off % N_DEV), N_DEV)

        def coff(d, r):
            return (-1 - r) if d == 0 else (1 + r)

        def cstart(d, off):
            return d * H + cidx(off) * R

        def cslice(ref, d, off):
            return ref.at[pl.ds(cstart(d, off), R)]

        def gather_issue(d, r):
            s = cstart(d, coff(d, r))

            def fn(t, c):
                owned = mask_i_ref[t] > 0

                @pl.when(owned)
                def _():
                    pltpu.make_async_copy(
                        e_ref.at[safe_ref[t]], gbuf.at[t],
                        gsem.at[2 * r + d],
                    ).start()

                return c + jnp.where(owned, 1, 0)

            return lax.fori_loop(s, s + R, fn, 0)

        def drain_convert(d, r, cnt):
            def fn(t, _):
                pltpu.make_async_copy(
                    e_ref.at[0], gbuf.at[0], gsem.at[2 * r + d]
                ).wait()
                return 0

            lax.fori_loop(0, cnt, fn, 0)
            s = cstart(d, coff(d, r))
            sl = pl.ds(s, R)
            red_ref[sl] = jnp.where(
                maskf_ref[sl] != 0, gbuf[sl].astype(jnp.bfloat16),
                jnp.bfloat16(0),
            )

        def oconv(d, off):
            sl = pl.ds(cstart(d, off), R)
            out_ref[sl] = red_ref[sl].astype(jnp.float32)

        cnt0 = [gather_issue(d, 0) for d in range(2)]

        barrier_sem = pltpu.get_barrier_semaphore()
        for nbr in (left, right):
            pl.semaphore_signal(
                barrier_sem, inc=1,
                device_id=(nbr,), device_id_type=pl.DeviceIdType.MESH,
            )
        pl.semaphore_wait(barrier_sem, 2)

        for d in range(2):
            drain_convert(d, 0, cnt0[d])

        for h in range(N_HOP):
            rdmas = []
            for d in range(2):
                rdma = pltpu.make_async_remote_copy(
                    src_ref=cslice(red_ref, d, coff(d, h)),
                    dst_ref=rs_buf.at[d, h],
                    send_sem=rs_send.at[d, h],
                    recv_sem=rs_recv.at[d, h],
                    device_id=(peer[d],),
                    device_id_type=pl.DeviceIdType.MESH,
                )
                rdma.start()
                rdmas.append(rdma)
            cnts = [gather_issue(d, h + 1) for d in range(2)]
            for d in range(2):
                drain_convert(d, h + 1, cnts[d])
            for d in range(2):
                rdmas[d].wait()
                dst = cslice(red_ref, d, coff(d, h + 1))
                dst[...] = dst[...] + rs_buf[d, h]

        for h in range(N_HOP):
            rdmas = []
            for d in range(2):
                s_off = -h if d == 0 else h
                rdma = pltpu.make_async_remote_copy(
                    src_ref=cslice(red_ref, d, s_off),
                    dst_ref=cslice(red_ref, d, s_off),
                    send_sem=ag_send.at[d, h],
                    recv_sem=ag_recv.at[d, h],
                    device_id=(peer[d],),
                    device_id_type=pl.DeviceIdType.MESH,
                )
                rdma.start()
                rdmas.append(rdma)
            for d in range(2):
                oconv(d, 0 if h == 0 else ((-h if d == 0 else h)))
            for rdma in rdmas:
                rdma.wait()
        for d in range(2):
            oconv(d, -N_HOP if d == 0 else N_HOP)

    return pl.pallas_call(
        body,
        out_shape=jax.ShapeDtypeStruct((T, D), jnp.float32),
        in_specs=[
            pl.BlockSpec(memory_space=pltpu.SMEM),
            pl.BlockSpec(memory_space=pltpu.SMEM),
            pl.BlockSpec(memory_space=pltpu.VMEM),
            pl.BlockSpec(memory_space=pl.ANY),
        ],
        out_specs=pl.BlockSpec(memory_space=pltpu.VMEM),
        scratch_shapes=[
            pltpu.VMEM((T, D), jnp.float32),
            pltpu.VMEM((T, D), jnp.bfloat16),
            pltpu.VMEM((2, N_HOP, R, D), jnp.bfloat16),
            pltpu.SemaphoreType.DMA((2 * N_DEV,)),
            pltpu.SemaphoreType.DMA((2, N_HOP)),
            pltpu.SemaphoreType.DMA((2, N_HOP)),
            pltpu.SemaphoreType.DMA((2, N_HOP)),
            pltpu.SemaphoreType.DMA((2, N_HOP)),
        ],
        compiler_params=pltpu.CompilerParams(collective_id=0),
    )(safe, mask_i, maskf, E)


# device time: 63154 ns/iter; 1.4195x vs baseline; 1.4195x over previous
import jax
import jax.numpy as jnp
from jax import lax
from jax.experimental import pallas as pl
from jax.experimental.pallas import tpu as pltpu

N_DEV = 4


def kernel(ids, E):
    T = ids.shape[0]
    V_per, D = E.shape
    H = T // 2
    Q = H // 4

    my = lax.axis_index("i")
    x0 = my // 2
    y0 = lax.rem((my + 1) // 2, 2)

    loc = ids - my * V_per
    mask = (loc >= 0) & (loc < V_per)
    safe = jnp.where(mask, loc, 0).astype(jnp.int32)
    maskf = mask.astype(jnp.bfloat16)[:, None]

    t_idx = jnp.arange(T, dtype=jnp.int32)
    blk = t_idx // (2 * Q)
    g = jnp.where(
        t_idx < H,
        jnp.where(blk == x0, 2, 0),
        jnp.where(blk - 2 == y0, 3, 1),
    )
    key = jnp.where(mask, g, 4)
    packed = jnp.sort(key * (1 << 25) + safe * (1 << 11) + t_idx)
    cum = jnp.cumsum(
        jnp.sum(jnp.where(key[None, :] == jnp.arange(4)[:, None], 1, 0), axis=1)
    ).astype(jnp.int32)

    def body(packed_ref, cum_ref, maskf_ref, e_ref, out_ref, gbuf, red_ref,
             rs1_buf, rs2_buf, gsem, p_send, p_recv):
        my_pos = lax.axis_index("i")
        xr = my_pos // 2
        yr = lax.rem((my_pos + 1) // 2, 2)
        xp = 3 - my_pos
        yp = my_pos + 1 - 2 * lax.rem(my_pos, 2)

        a_send = (1 - xr) * 2 * Q
        a_keep = xr * 2 * Q
        b_send = H + (1 - yr) * 2 * Q
        b_keep = H + yr * 2 * Q
        qa_keep = my_pos * Q
        qa_send = (4 * xr + 1 - my_pos) * Q
        qb_keep = H + (2 * yr + xr) * Q
        qb_send = H + (2 * yr + 1 - xr) * Q

        def issue_seg(seg, lo, hi):
            def fn(t, _):
                v = packed_ref[t]
                pltpu.make_async_copy(
                    e_ref.at[(v >> 11) & (16 * 1024 - 1)],
                    gbuf.at[v & (2 * 1024 - 1)],
                    gsem.at[seg],
                ).start()
                return 0

            lax.fori_loop(lo, hi, fn, 0)

        def drain_convert(seg, lo, hi, start):
            def fn(t, _):
                pltpu.make_async_copy(
                    e_ref.at[0], gbuf.at[0], gsem.at[seg]
                ).wait()
                return 0

            lax.fori_loop(0, hi - lo, fn, 0)
            sl = pl.ds(start, 2 * Q)
            red_ref[sl] = jnp.where(
                maskf_ref[sl] != 0, gbuf[sl].astype(jnp.bfloat16),
                jnp.bfloat16(0),
            )

        def exchange(ph, srcs, dsts, peers, n):
            rdmas = []
            for k in range(2):
                rdma = pltpu.make_async_remote_copy(
                    src_ref=red_ref.at[pl.ds(srcs[k], n)],
                    dst_ref=dsts[k],
                    send_sem=p_send.at[ph, k],
                    recv_sem=p_recv.at[ph, k],
                    device_id=(peers[k],),
                    device_id_type=pl.DeviceIdType.MESH,
                )
                rdma.start()
                rdmas.append(rdma)
            return rdmas

        def accum(start, buf, n):
            sl = pl.ds(start, n)
            red_ref[sl] = red_ref[sl] + buf

        def oconv(start, n):
            sl = pl.ds(start, n)
            out_ref[sl] = red_ref[sl].astype(jnp.float32)

        issue_seg(0, 0, cum_ref[0])
        issue_seg(1, cum_ref[0], cum_ref[1])

        barrier_sem = pltpu.get_barrier_semaphore()
        for nbr in (xp, yp):
            pl.semaphore_signal(
                barrier_sem, inc=1,
                device_id=(nbr,), device_id_type=pl.DeviceIdType.MESH,
            )
        pl.semaphore_wait(barrier_sem, 2)

        drain_convert(0, 0, cum_ref[0], a_send)
        drain_convert(1, cum_ref[0], cum_ref[1], b_send)

        ph1 = exchange(
            0, (a_send, b_send),
            (rs1_buf.at[0], rs1_buf.at[1]), (xp, yp), 2 * Q,
        )
        issue_seg(2, cum_ref[1], cum_ref[2])
        issue_seg(3, cum_ref[2], cum_ref[3])
        drain_convert(2, cum_ref[1], cum_ref[2], a_keep)
        drain_convert(3, cum_ref[2], cum_ref[3], b_keep)
        for r in ph1:
            r.wait()
        accum(a_keep, rs1_buf[0], 2 * Q)
        accum(b_keep, rs1_buf[1], 2 * Q)

        ph2 = exchange(
            1, (qa_send, qb_send),
            (rs2_buf.at[0], rs2_buf.at[1]), (yp, xp), Q,
        )
        for r in ph2:
            r.wait()
        accum(qa_keep, rs2_buf[0], Q)
        accum(qb_keep, rs2_buf[1], Q)

        ph3 = exchange(
            2, (qa_keep, qb_keep),
            (red_ref.at[pl.ds(qa_keep, Q)], red_ref.at[pl.ds(qb_keep, Q)]),
            (yp, xp), Q,
        )
        oconv(qa_keep, Q)
        oconv(qb_keep, Q)
        for r in ph3:
            r.wait()
        oconv(qa_send, Q)
        oconv(qb_send, Q)

        ph4 = exchange(
            3, (a_keep, b_keep),
            (red_ref.at[pl.ds(a_keep, 2 * Q)], red_ref.at[pl.ds(b_keep, 2 * Q)]),
            (xp, yp), 2 * Q,
        )
        for r in ph4:
            r.wait()
        oconv(a_send, 2 * Q)
        oconv(b_send, 2 * Q)

    return pl.pallas_call(
        body,
        out_shape=jax.ShapeDtypeStruct((T, D), jnp.float32),
        in_specs=[
            pl.BlockSpec(memory_space=pltpu.SMEM),
            pl.BlockSpec(memory_space=pltpu.SMEM),
            pl.BlockSpec(memory_space=pltpu.VMEM),
            pl.BlockSpec(memory_space=pl.ANY),
        ],
        out_specs=pl.BlockSpec(memory_space=pltpu.VMEM),
        scratch_shapes=[
            pltpu.VMEM((T, D), jnp.float32),
            pltpu.VMEM((T, D), jnp.bfloat16),
            pltpu.VMEM((2, 2 * Q, D), jnp.bfloat16),
            pltpu.VMEM((2, Q, D), jnp.bfloat16),
            pltpu.SemaphoreType.DMA((4,)),
            pltpu.SemaphoreType.DMA((4, 2)),
            pltpu.SemaphoreType.DMA((4, 2)),
        ],
        compiler_params=pltpu.CompilerParams(collective_id=0),
    )(packed, cum, maskf, E)


# device time: 58202 ns/iter; 1.5402x vs baseline; 1.0851x over previous
import jax
import jax.numpy as jnp
from jax import lax
from jax.experimental import pallas as pl
from jax.experimental.pallas import tpu as pltpu

N_DEV = 4


def kernel(ids, E):
    T = ids.shape[0]
    V_per, D = E.shape
    H = T // 2
    Q = H // 4

    my = lax.axis_index("i")
    x0 = my // 2
    y0 = lax.rem((my + 1) // 2, 2)

    loc = ids - my * V_per
    mask = (loc >= 0) & (loc < V_per)
    safe = jnp.where(mask, loc, 0).astype(jnp.int32)
    maskf = mask.astype(jnp.bfloat16)[:, None]

    t_idx = jnp.arange(T, dtype=jnp.int32)
    blk = t_idx // (2 * Q)
    g = jnp.where(
        t_idx < H,
        jnp.where(blk == x0, 2, 0),
        jnp.where(blk - 2 == y0, 3, 1),
    )
    key = jnp.where(mask, g, 4)
    packed = jnp.sort(key * (1 << 25) + safe * (1 << 11) + t_idx)
    cum = jnp.cumsum(
        jnp.sum(jnp.where(key[None, :] == jnp.arange(4)[:, None], 1, 0), axis=1)
    ).astype(jnp.int32)

    def body(packed_ref, cum_ref, maskf_ref, e_ref, out_ref, gbuf, red_ref,
             rs1_buf, rs2_buf, gsem, p_send, p_recv):
        my_pos = lax.axis_index("i")
        xr = my_pos // 2
        yr = lax.rem((my_pos + 1) // 2, 2)
        xp = 3 - my_pos
        yp = my_pos + 1 - 2 * lax.rem(my_pos, 2)

        a_send = (1 - xr) * 2 * Q
        a_keep = xr * 2 * Q
        b_send = H + (1 - yr) * 2 * Q
        b_keep = H + yr * 2 * Q
        keep = (a_keep, b_keep)
        send = (a_send, b_send)
        partners = ((xp, yp), (yp, xp), (xp, yp))

        def issue_seg(seg, lo, hi):
            def fn(t, _):
                v = packed_ref[t]
                pltpu.make_async_copy(
                    e_ref.at[(v >> 11) & (16 * 1024 - 1)],
                    gbuf.at[v & (2 * 1024 - 1)],
                    gsem.at[seg],
                ).start()
                return 0

            lax.fori_loop(lo, hi, fn, 0)

        def drain_convert(seg, lo, hi, start):
            def fn(t, _):
                pltpu.make_async_copy(
                    e_ref.at[0], gbuf.at[0], gsem.at[seg]
                ).wait()
                return 0

            lax.fori_loop(0, hi - lo, fn, 0)
            sl = pl.ds(start, 2 * Q)
            red_ref[sl] = jnp.where(
                maskf_ref[sl] != 0, gbuf[sl].astype(jnp.bfloat16),
                jnp.bfloat16(0),
            )

        def start_piece(ph, half, pc, src_row, dst):
            rdma = pltpu.make_async_remote_copy(
                src_ref=red_ref.at[pl.ds(src_row, Q)],
                dst_ref=dst,
                send_sem=p_send.at[ph, half, pc],
                recv_sem=p_recv.at[ph, half, pc],
                device_id=(partners[ph][half],),
                device_id_type=pl.DeviceIdType.MESH,
            )
            rdma.start()
            return rdma

        def accum(start, buf):
            sl = pl.ds(start, Q)
            red_ref[sl] = red_ref[sl] + buf

        def oconv(start):
            sl = pl.ds(start, Q)
            out_ref[sl] = red_ref[sl].astype(jnp.float32)

        issue_seg(0, 0, cum_ref[0])
        issue_seg(1, cum_ref[0], cum_ref[1])

        barrier_sem = pltpu.get_barrier_semaphore()
        for nbr in (xp, yp):
            pl.semaphore_signal(
                barrier_sem, inc=1,
                device_id=(nbr,), device_id_type=pl.DeviceIdType.MESH,
            )
        pl.semaphore_wait(barrier_sem, 2)

        drain_convert(0, 0, cum_ref[0], a_send)
        p1 = {}
        for pc in range(2):
            p1[0, pc] = start_piece(0, 0, pc, a_send + pc * Q,
                                    rs1_buf.at[0, pc])
        drain_convert(1, cum_ref[0], cum_ref[1], b_send)
        for pc in range(2):
            p1[1, pc] = start_piece(0, 1, pc, b_send + pc * Q,
                                    rs1_buf.at[1, pc])

        issue_seg(2, cum_ref[1], cum_ref[2])
        issue_seg(3, cum_ref[2], cum_ref[3])
        drain_convert(2, cum_ref[1], cum_ref[2], a_keep)
        drain_convert(3, cum_ref[2], cum_ref[3], b_keep)

        p2 = {}
        for pc in range(2):
            for half in range(2):
                p1[half, pc].wait()
                accum(keep[half] + pc * Q, rs1_buf[half, pc])
                p2[half, pc] = start_piece(1, half, pc, keep[half] + pc * Q,
                                           rs2_buf.at[half, pc])

        p3 = {}
        for pc in range(2):
            for half in range(2):
                p2[half, pc].wait()
                accum(keep[half] + pc * Q, rs2_buf[half, pc])
                p3[half, pc] = start_piece(
                    2, half, pc, keep[half] + pc * Q,
                    red_ref.at[pl.ds(keep[half] + pc * Q, Q)],
                )
                oconv(keep[half] + pc * Q)

        for pc in range(2):
            for half in range(2):
                p3[half, pc].wait()
                oconv(send[half] + pc * Q)

    return pl.pallas_call(
        body,
        out_shape=jax.ShapeDtypeStruct((T, D), jnp.float32),
        in_specs=[
            pl.BlockSpec(memory_space=pltpu.SMEM),
            pl.BlockSpec(memory_space=pltpu.SMEM),
            pl.BlockSpec(memory_space=pltpu.VMEM),
            pl.BlockSpec(memory_space=pl.ANY),
        ],
        out_specs=pl.BlockSpec(memory_space=pltpu.VMEM),
        scratch_shapes=[
            pltpu.VMEM((T, D), jnp.float32),
            pltpu.VMEM((T, D), jnp.bfloat16),
            pltpu.VMEM((2, 2, Q, D), jnp.bfloat16),
            pltpu.VMEM((2, 2, Q, D), jnp.bfloat16),
            pltpu.SemaphoreType.DMA((4,)),
            pltpu.SemaphoreType.DMA((3, 2, 2)),
            pltpu.SemaphoreType.DMA((3, 2, 2)),
        ],
        compiler_params=pltpu.CompilerParams(collective_id=0),
    )(packed, cum, maskf, E)


# device time: 56350 ns/iter; 1.5909x vs baseline; 1.0329x over previous
import jax
import jax.numpy as jnp
from jax import lax
from jax.experimental import pallas as pl
from jax.experimental.pallas import tpu as pltpu

N_DEV = 4


def kernel(ids, E):
    T = ids.shape[0]
    V_per, D = E.shape
    H = T // 2
    Q = H // 4

    my = lax.axis_index("i")
    x0 = my // 2
    y0 = lax.rem((my + 1) // 2, 2)

    loc = ids - my * V_per
    mask = (loc >= 0) & (loc < V_per)
    safe = jnp.where(mask, loc, 0).astype(jnp.int32)
    maskf = mask.astype(jnp.bfloat16)[:, None]

    t_idx = jnp.arange(T, dtype=jnp.int32)
    blk = t_idx // (2 * Q)
    is_keep = jnp.where(t_idx < H, blk == x0, blk - 2 == y0)
    piece = lax.rem(t_idx // Q, 2)
    is_b = (t_idx >= H).astype(jnp.int32)
    seg = jnp.where(is_keep, 4, 0) + piece * 2 + is_b
    key = jnp.where(mask, seg, 8)
    packed = jnp.sort(key * (1 << 25) + safe * (1 << 11) + t_idx)
    cum = jnp.cumsum(
        jnp.sum(jnp.where(key[None, :] == jnp.arange(8)[:, None], 1, 0), axis=1)
    ).astype(jnp.int32)

    def body(packed_ref, cum_ref, maskf_ref, e_ref, out_ref, gbuf, red_ref,
             ostage, rs1_buf, rs2_buf, gsem, osem, p_send, p_recv):
        my_pos = lax.axis_index("i")
        xr = my_pos // 2
        yr = lax.rem((my_pos + 1) // 2, 2)
        xp = 3 - my_pos
        yp = my_pos + 1 - 2 * lax.rem(my_pos, 2)

        a_send = (1 - xr) * 2 * Q
        a_keep = xr * 2 * Q
        b_send = H + (1 - yr) * 2 * Q
        b_keep = H + yr * 2 * Q
        keep = (a_keep, b_keep)
        send = (a_send, b_send)
        partners = ((xp, yp), (yp, xp), (xp, yp))

        def issue_seg(s, lo, hi):
            def fn(t, _):
                v = packed_ref[t]
                pltpu.make_async_copy(
                    e_ref.at[(v >> 11) & (16 * 1024 - 1)],
                    gbuf.at[v & (2 * 1024 - 1)],
                    gsem.at[s],
                ).start()
                return 0

            lax.fori_loop(lo, hi, fn, 0)

        def drain_convert(s, lo, hi, start):
            def fn(t, _):
                pltpu.make_async_copy(
                    e_ref.at[0], gbuf.at[0], gsem.at[s]
                ).wait()
                return 0

            lax.fori_loop(0, hi - lo, fn, 0)
            sl = pl.ds(start, Q)
            red_ref[sl] = jnp.where(
                maskf_ref[sl] != 0, gbuf[sl].astype(jnp.bfloat16),
                jnp.bfloat16(0),
            )

        def start_piece(ph, half, pc, src_row, dst):
            rdma = pltpu.make_async_remote_copy(
                src_ref=red_ref.at[pl.ds(src_row, Q)],
                dst_ref=dst,
                send_sem=p_send.at[ph, half, pc],
                recv_sem=p_recv.at[ph, half, pc],
                device_id=(partners[ph][half],),
                device_id_type=pl.DeviceIdType.MESH,
            )
            rdma.start()
            return rdma

        def accum(start, buf):
            sl = pl.ds(start, Q)
            red_ref[sl] = red_ref[sl] + buf

        def oflush(start):
            sl = pl.ds(start, Q)
            ostage[sl] = red_ref[sl].astype(jnp.float32)
            pltpu.make_async_copy(ostage.at[sl], out_ref.at[sl], osem).start()

        for s in range(4):
            issue_seg(s, 0 if s == 0 else cum_ref[s - 1], cum_ref[s])

        barrier_sem = pltpu.get_barrier_semaphore()
        for nbr in (xp, yp):
            pl.semaphore_signal(
                barrier_sem, inc=1,
                device_id=(nbr,), device_id_type=pl.DeviceIdType.MESH,
            )
        pl.semaphore_wait(barrier_sem, 2)

        p1 = {}
        for s, (half, pc) in enumerate(((0, 0), (1, 0), (0, 1), (1, 1))):
            drain_convert(s, 0 if s == 0 else cum_ref[s - 1],
                          cum_ref[s], send[half] + pc * Q)
            p1[half, pc] = start_piece(0, half, pc, send[half] + pc * Q,
                                       rs1_buf.at[half, pc])

        for s in range(4, 8):
            issue_seg(s, cum_ref[s - 1], cum_ref[s])
        for s, (half, pc) in enumerate(((0, 0), (1, 0), (0, 1), (1, 1))):
            drain_convert(s + 4, cum_ref[s + 3], cum_ref[s + 4],
                          keep[half] + pc * Q)

        p2 = {}
        for pc in range(2):
            for half in range(2):
                p1[half, pc].wait()
                accum(keep[half] + pc * Q, rs1_buf[half, pc])
                p2[half, pc] = start_piece(1, half, pc, keep[half] + pc * Q,
                                           rs2_buf.at[half, pc])

        p3 = {}
        for pc in range(2):
            for half in range(2):
                p2[half, pc].wait()
                accum(keep[half] + pc * Q, rs2_buf[half, pc])
                p3[half, pc] = start_piece(
                    2, half, pc, keep[half] + pc * Q,
                    red_ref.at[pl.ds(keep[half] + pc * Q, Q)],
                )
                oflush(keep[half] + pc * Q)

        for pc in range(2):
            for half in range(2):
                p3[half, pc].wait()
                oflush(send[half] + pc * Q)

        for _ in range(8):
            pltpu.make_async_copy(
                ostage.at[pl.ds(0, Q)], out_ref.at[pl.ds(0, Q)], osem
            ).wait()

    return pl.pallas_call(
        body,
        out_shape=jax.ShapeDtypeStruct((T, D), jnp.float32),
        in_specs=[
            pl.BlockSpec(memory_space=pltpu.SMEM),
            pl.BlockSpec(memory_space=pltpu.SMEM),
            pl.BlockSpec(memory_space=pltpu.VMEM),
            pl.BlockSpec(memory_space=pl.ANY),
        ],
        out_specs=pl.BlockSpec(memory_space=pl.ANY),
        scratch_shapes=[
            pltpu.VMEM((T, D), jnp.float32),
            pltpu.VMEM((T, D), jnp.bfloat16),
            pltpu.VMEM((T, D), jnp.float32),
            pltpu.VMEM((2, 2, Q, D), jnp.bfloat16),
            pltpu.VMEM((2, 2, Q, D), jnp.bfloat16),
            pltpu.SemaphoreType.DMA((8,)),
            pltpu.SemaphoreType.DMA,
            pltpu.SemaphoreType.DMA((3, 2, 2)),
            pltpu.SemaphoreType.DMA((3, 2, 2)),
        ],
        compiler_params=pltpu.CompilerParams(collective_id=0),
    )(packed, cum, maskf, E)
